# baseline (speedup 1.0000x reference)
import functools

import jax
import jax.numpy as jnp
from jax import lax
from jax.experimental import pallas as pl
from jax.experimental.pallas import tpu as pltpu
from jax.experimental.pallas import tpu_sc as plsc

N = 10000
D = 128
E = 320000
CHUNK = 128
NC = 2
NS = 16
NW = NC * NS
SB = 10
NSB = 8
CPT = SB * NSB
NUM_CHUNKS = NW * CPT
E_PAD = NUM_CHUNKS * CHUNK
PAD = E_PAD - E
DEPTH = 2
NPAD = N + 8
RT = 624
RT_LAST = N - (NS - 1) * RT


@functools.cache
def _build_agg():
    mesh = plsc.VectorSubcoreMesh(core_axis_name="c", subcore_axis_name="s")

    def body(h_hbm, src_hbm, dst_hbm, z_hbm, psum_hbm,
             srci, dsti, rows, acc_sh, gsem, ssem):
        cid = lax.axis_index("c")
        sid = lax.axis_index("s")
        gwid = cid * NS + sid

        @pl.when(sid < NS - 1)
        def _():
            pltpu.sync_copy(z_hbm.at[pl.ds(0, RT)],
                            acc_sh.at[pl.ds(sid * RT, RT)])

        @pl.when(sid == NS - 1)
        def _():
            pltpu.sync_copy(z_hbm, acc_sh.at[pl.ds((NS - 1) * RT, RT_LAST)])

        plsc.subcore_barrier()

        def gather_start(j, r):
            pltpu.async_copy(h_hbm.at[srci.at[j]], rows.at[r], gsem.at[r])

        def gather_wait(j, r):
            pltpu.make_async_copy(h_hbm.at[srci.at[j]], rows.at[r],
                                  gsem.at[r]).wait()

        def scat_start(j, r):
            pltpu.async_copy(rows.at[r], acc_sh.at[dsti.at[j]], ssem.at[r],
                             add=True)

        def scat_wait(j, r):
            pltpu.make_async_copy(rows.at[r], acc_sh.at[dsti.at[j]],
                                  ssem.at[r]).wait()

        @pl.loop(0, NSB)
        def _(s):
            pltpu.sync_copy(src_hbm.at[gwid, s], srci)
            pltpu.sync_copy(dst_hbm.at[gwid, s], dsti)
            for j in range(SB):
                r = j % DEPTH
                if j >= DEPTH:
                    scat_wait(j - DEPTH, r)
                gather_start(j, r)
                if j >= 1:
                    rp = (j - 1) % DEPTH
                    gather_wait(j - 1, rp)
                    scat_start(j - 1, rp)
            rl = (SB - 1) % DEPTH
            gather_wait(SB - 1, rl)
            scat_start(SB - 1, rl)
            for j in range(SB - DEPTH, SB):
                scat_wait(j, j % DEPTH)

        plsc.subcore_barrier()

        @pl.when(sid < NS - 1)
        def _():
            b = sid * RT
            pltpu.sync_copy(acc_sh.at[pl.ds(b, RT)],
                            psum_hbm.at[pl.ds(cid * N + b, RT)])

        @pl.when(sid == NS - 1)
        def _():
            b = (NS - 1) * RT
            pltpu.sync_copy(acc_sh.at[pl.ds(b, RT_LAST)],
                            psum_hbm.at[pl.ds(cid * N + b, RT_LAST)])

    return pl.kernel(
        body,
        out_type=jax.ShapeDtypeStruct((NC * N, D), jnp.float32),
        mesh=mesh,
        scratch_types=[
            pltpu.VMEM((SB, CHUNK), jnp.int32),
            pltpu.VMEM((SB, CHUNK), jnp.int32),
            pltpu.VMEM((DEPTH, CHUNK, D), jnp.float32),
            pltpu.VMEM_SHARED((NPAD, D), jnp.float32),
            pltpu.SemaphoreType.DMA((DEPTH,)),
            pltpu.SemaphoreType.DMA((DEPTH,)),
        ],
    )


@functools.cache
def _build_counts():
    mesh = plsc.VectorSubcoreMesh(core_axis_name="c", subcore_axis_name="s")

    def body(dst_hbm, z_hbm, ones_hbm, pcnt_hbm, dsti, ones_v, cnt_sh, sem):
        cid = lax.axis_index("c")
        sid = lax.axis_index("s")
        gwid = cid * NS + sid

        @pl.when(sid < NS - 1)
        def _():
            pltpu.sync_copy(z_hbm.at[pl.ds(0, RT)],
                            cnt_sh.at[pl.ds(sid * RT, RT)])

        @pl.when(sid == NS - 1)
        def _():
            pltpu.sync_copy(z_hbm, cnt_sh.at[pl.ds((NS - 1) * RT, RT_LAST)])

        pltpu.sync_copy(ones_hbm, ones_v)
        plsc.subcore_barrier()

        @pl.loop(0, NSB)
        def _(s):
            pltpu.sync_copy(dst_hbm.at[gwid, s], dsti)
            for j in range(SB):
                pltpu.async_copy(ones_v, cnt_sh.at[dsti.at[j]], sem, add=True)
            for j in range(SB):
                pltpu.make_async_copy(ones_v, cnt_sh.at[dsti.at[j]],
                                      sem).wait()

        plsc.subcore_barrier()

        @pl.when(sid < NS - 1)
        def _():
            b = sid * RT
            pltpu.sync_copy(cnt_sh.at[pl.ds(b, RT)],
                            pcnt_hbm.at[pl.ds(cid * N + b, RT)])

        @pl.when(sid == NS - 1)
        def _():
            b = (NS - 1) * RT
            pltpu.sync_copy(cnt_sh.at[pl.ds(b, RT_LAST)],
                            pcnt_hbm.at[pl.ds(cid * N + b, RT_LAST)])

    return pl.kernel(
        body,
        out_type=jax.ShapeDtypeStruct((NC * N, D), jnp.float32),
        mesh=mesh,
        scratch_types=[
            pltpu.VMEM((SB, CHUNK), jnp.int32),
            pltpu.VMEM((CHUNK, D), jnp.float32),
            pltpu.VMEM_SHARED((NPAD, D), jnp.float32),
            pltpu.SemaphoreType.DMA,
        ],
    )


BN = 2000


@functools.cache
def _build_linear(relu: bool):
    def body(h_ref, ps_ref, pc_ref, wt_ref, b_ref, o_ref):
        sums = ps_ref[0] + ps_ref[1]
        cnt = pc_ref[0][:, 0:1] + pc_ref[1][:, 0:1]
        row0 = pl.program_id(0) * BN
        rows = jax.lax.broadcasted_iota(jnp.int32, (BN, 1), 0) + row0
        cnt = cnt - jnp.where(rows < PAD, 1.0, 0.0)
        h_n = sums / jnp.maximum(cnt, 1.0)
        acc = jnp.dot(h_ref[...], wt_ref[0:D, :],
                      preferred_element_type=jnp.float32)
        acc += jnp.dot(h_n, wt_ref[D:2 * D, :],
                       preferred_element_type=jnp.float32)
        acc += b_ref[...]
        if relu:
            acc = jnp.maximum(acc, 0.0)
        o_ref[...] = acc

    return pl.pallas_call(
        body,
        grid=(N // BN,),
        in_specs=[
            pl.BlockSpec((BN, D), lambda i: (i, 0)),
            pl.BlockSpec((NC, BN, D), lambda i: (0, i, 0)),
            pl.BlockSpec((NC, BN, D), lambda i: (0, i, 0)),
            pl.BlockSpec((2 * D, D), lambda i: (0, 0)),
            pl.BlockSpec((1, D), lambda i: (0, 0)),
        ],
        out_specs=pl.BlockSpec((BN, D), lambda i: (i, 0)),
        out_shape=jax.ShapeDtypeStruct((N, D), jnp.float32),
    )


def kernel(x, edge_index, W1, b1, W2, b2):
    pad_src = N + (jnp.arange(PAD, dtype=jnp.int32) % (NPAD - N))
    pad_dst = jnp.arange(PAD, dtype=jnp.int32)
    src = jnp.concatenate(
        [edge_index[0].reshape(NW, E // NW),
         pad_src.reshape(NW, PAD // NW)], axis=1
    ).reshape(NW, NSB, SB, CHUNK)
    dst = jnp.concatenate(
        [edge_index[1].reshape(NW, E // NW),
         pad_dst.reshape(NW, PAD // NW)], axis=1
    ).reshape(NW, NSB, SB, CHUNK)
    zeros = jnp.zeros((RT_LAST, D), jnp.float32)
    ones = jnp.ones((CHUNK, D), jnp.float32)
    w1t = W1.T
    w2t = W2.T
    b1r = b1.reshape(1, D)
    b2r = b2.reshape(1, D)

    zrow = jnp.zeros((NPAD - N, D), jnp.float32)
    pc = _build_counts()(dst, zeros, ones).reshape(NC, N, D)
    ps1 = _build_agg()(jnp.concatenate([x, zrow]), src, dst,
                       zeros).reshape(NC, N, D)
    h1 = _build_linear(True)(x, ps1, pc, w1t, b1r)
    ps2 = _build_agg()(jnp.concatenate([h1, zrow]), src, dst,
                       zeros).reshape(NC, N, D)
    return _build_linear(False)(h1, ps2, pc, w2t, b2r)


# speedup vs baseline: 1.1833x; 1.1833x over previous
import functools

import jax
import jax.numpy as jnp
from jax import lax
from jax.experimental import pallas as pl
from jax.experimental.pallas import tpu as pltpu
from jax.experimental.pallas import tpu_sc as plsc

N = 10000
D = 128
E = 320000
CHUNK = 128
NC = 2
NS = 16
NW = NC * NS
SB = 16
NSB = 5
CPT = SB * NSB
NUM_CHUNKS = NW * CPT
E_PAD = NUM_CHUNKS * CHUNK
PAD = E_PAD - E
DEPTH = 2
NPAD = N + 64
RT = 624
RT_LAST = N - (NS - 1) * RT


@functools.cache
def _build_agg():
    mesh = plsc.VectorSubcoreMesh(core_axis_name="c", subcore_axis_name="s")

    def body(h_hbm, src_hbm, dst_hbm, z_hbm, psum_hbm,
             srci, dsti, rows, acc_sh, gsem, ssem):
        cid = lax.axis_index("c")
        sid = lax.axis_index("s")
        gwid = cid * NS + sid

        @pl.when(sid < NS - 1)
        def _():
            pltpu.sync_copy(z_hbm.at[pl.ds(0, RT)],
                            acc_sh.at[pl.ds(sid * RT, RT)])

        @pl.when(sid == NS - 1)
        def _():
            pltpu.sync_copy(z_hbm, acc_sh.at[pl.ds((NS - 1) * RT, RT_LAST)])

        plsc.subcore_barrier()

        def gather_start(j, r):
            pltpu.async_copy(h_hbm.at[srci.at[j]], rows.at[r], gsem.at[r])

        def gather_wait(j, r):
            pltpu.make_async_copy(h_hbm.at[srci.at[j]], rows.at[r],
                                  gsem.at[r]).wait()

        def scat_start(j, r):
            pltpu.async_copy(rows.at[r], acc_sh.at[dsti.at[j]], ssem.at[r],
                             add=True)

        def scat_wait(j, r):
            pltpu.make_async_copy(rows.at[r], acc_sh.at[dsti.at[j]],
                                  ssem.at[r]).wait()

        @pl.loop(0, NSB)
        def _(s):
            pltpu.sync_copy(src_hbm.at[gwid, s], srci)
            pltpu.sync_copy(dst_hbm.at[gwid, s], dsti)
            for j in range(SB):
                r = j % DEPTH
                if j >= DEPTH:
                    scat_wait(j - DEPTH, r)
                gather_start(j, r)
                if j >= 1:
                    rp = (j - 1) % DEPTH
                    gather_wait(j - 1, rp)
                    scat_start(j - 1, rp)
            rl = (SB - 1) % DEPTH
            gather_wait(SB - 1, rl)
            scat_start(SB - 1, rl)
            for j in range(SB - DEPTH, SB):
                scat_wait(j, j % DEPTH)

        plsc.subcore_barrier()

        @pl.when(sid < NS - 1)
        def _():
            b = sid * RT
            pltpu.sync_copy(acc_sh.at[pl.ds(b, RT)],
                            psum_hbm.at[pl.ds(cid * N + b, RT)])

        @pl.when(sid == NS - 1)
        def _():
            b = (NS - 1) * RT
            pltpu.sync_copy(acc_sh.at[pl.ds(b, RT_LAST)],
                            psum_hbm.at[pl.ds(cid * N + b, RT_LAST)])

    return pl.kernel(
        body,
        out_type=jax.ShapeDtypeStruct((NC * N, D), jnp.float32),
        mesh=mesh,
        scratch_types=[
            pltpu.VMEM((SB, CHUNK), jnp.int32),
            pltpu.VMEM((SB, CHUNK), jnp.int32),
            pltpu.VMEM((DEPTH, CHUNK, D), jnp.float32),
            pltpu.VMEM_SHARED((NPAD, D), jnp.float32),
            pltpu.SemaphoreType.DMA((DEPTH,)),
            pltpu.SemaphoreType.DMA((DEPTH,)),
        ],
    )


@functools.cache
def _build_counts():
    mesh = plsc.VectorSubcoreMesh(core_axis_name="c", subcore_axis_name="s")

    def body(dst_hbm, z_hbm, ones_hbm, pcnt_hbm, dsti, ones_v, cnt_sh, sem):
        cid = lax.axis_index("c")
        sid = lax.axis_index("s")
        gwid = cid * NS + sid

        @pl.when(sid < NS - 1)
        def _():
            pltpu.sync_copy(z_hbm.at[pl.ds(0, RT)],
                            cnt_sh.at[pl.ds(sid * RT, RT)])

        @pl.when(sid == NS - 1)
        def _():
            pltpu.sync_copy(z_hbm, cnt_sh.at[pl.ds((NS - 1) * RT, RT_LAST)])

        pltpu.sync_copy(ones_hbm, ones_v)
        plsc.subcore_barrier()

        @pl.loop(0, NSB)
        def _(s):
            pltpu.sync_copy(dst_hbm.at[gwid, s], dsti)
            for j in range(SB):
                pltpu.async_copy(ones_v, cnt_sh.at[dsti.at[j]], sem, add=True)
            for j in range(SB):
                pltpu.make_async_copy(ones_v, cnt_sh.at[dsti.at[j]],
                                      sem).wait()

        plsc.subcore_barrier()

        @pl.when(sid < NS - 1)
        def _():
            b = sid * RT
            pltpu.sync_copy(cnt_sh.at[pl.ds(b, RT)],
                            pcnt_hbm.at[pl.ds(cid * N + b, RT)])

        @pl.when(sid == NS - 1)
        def _():
            b = (NS - 1) * RT
            pltpu.sync_copy(cnt_sh.at[pl.ds(b, RT_LAST)],
                            pcnt_hbm.at[pl.ds(cid * N + b, RT_LAST)])

    return pl.kernel(
        body,
        out_type=jax.ShapeDtypeStruct((NC * N, D), jnp.float32),
        mesh=mesh,
        scratch_types=[
            pltpu.VMEM((SB, CHUNK), jnp.int32),
            pltpu.VMEM((CHUNK, D), jnp.float32),
            pltpu.VMEM_SHARED((NPAD, D), jnp.float32),
            pltpu.SemaphoreType.DMA,
        ],
    )


BN = 2000


@functools.cache
def _build_linear(relu: bool):
    def body(h_ref, ps_ref, pc_ref, wt_ref, b_ref, o_ref):
        sums = ps_ref[0] + ps_ref[1]
        cnt = pc_ref[0][:, 0:1] + pc_ref[1][:, 0:1]
        row0 = pl.program_id(0) * BN
        rows = jax.lax.broadcasted_iota(jnp.int32, (BN, 1), 0) + row0
        cnt = cnt - jnp.where(rows < PAD, 1.0, 0.0)
        h_n = sums / jnp.maximum(cnt, 1.0)
        acc = jnp.dot(h_ref[...], wt_ref[0:D, :],
                      preferred_element_type=jnp.float32)
        acc += jnp.dot(h_n, wt_ref[D:2 * D, :],
                       preferred_element_type=jnp.float32)
        acc += b_ref[...]
        if relu:
            acc = jnp.maximum(acc, 0.0)
        o_ref[...] = acc

    return pl.pallas_call(
        body,
        grid=(N // BN,),
        in_specs=[
            pl.BlockSpec((BN, D), lambda i: (i, 0)),
            pl.BlockSpec((NC, BN, D), lambda i: (0, i, 0)),
            pl.BlockSpec((NC, BN, D), lambda i: (0, i, 0)),
            pl.BlockSpec((2 * D, D), lambda i: (0, 0)),
            pl.BlockSpec((1, D), lambda i: (0, 0)),
        ],
        out_specs=pl.BlockSpec((BN, D), lambda i: (i, 0)),
        out_shape=jax.ShapeDtypeStruct((N, D), jnp.float32),
    )


def kernel(x, edge_index, W1, b1, W2, b2):
    pad_src = N + (jnp.arange(PAD, dtype=jnp.int32) % (NPAD - N))
    pad_dst = jnp.arange(PAD, dtype=jnp.int32)
    src = jnp.concatenate(
        [edge_index[0].reshape(NW, E // NW),
         pad_src.reshape(NW, PAD // NW)], axis=1
    ).reshape(NW, NSB, SB, CHUNK)
    dst = jnp.concatenate(
        [edge_index[1].reshape(NW, E // NW),
         pad_dst.reshape(NW, PAD // NW)], axis=1
    ).reshape(NW, NSB, SB, CHUNK)
    zeros = jnp.zeros((RT_LAST, D), jnp.float32)
    ones = jnp.ones((CHUNK, D), jnp.float32)
    w1t = W1.T
    w2t = W2.T
    b1r = b1.reshape(1, D)
    b2r = b2.reshape(1, D)

    zrow = jnp.zeros((NPAD - N, D), jnp.float32)
    pc = _build_counts()(dst, zeros, ones).reshape(NC, N, D)
    ps1 = _build_agg()(jnp.concatenate([x, zrow]), src, dst,
                       zeros).reshape(NC, N, D)
    h1 = _build_linear(True)(x, ps1, pc, w1t, b1r)
    ps2 = _build_agg()(jnp.concatenate([h1, zrow]), src, dst,
                       zeros).reshape(NC, N, D)
    return _build_linear(False)(h1, ps2, pc, w2t, b2r)


# speedup vs baseline: 1.1867x; 1.0028x over previous
import functools

import jax
import jax.numpy as jnp
from jax import lax
from jax.experimental import pallas as pl
from jax.experimental.pallas import tpu as pltpu
from jax.experimental.pallas import tpu_sc as plsc

N = 10000
D = 128
E = 320000
CHUNK = 128
NC = 2
NS = 16
NW = NC * NS
SB = 16
NSB = 5
CPT = SB * NSB
NUM_CHUNKS = NW * CPT
E_PAD = NUM_CHUNKS * CHUNK
PAD = E_PAD - E
DEPTH = 2
NPAD = N + 64
CW = 16
RT = 624
RT_LAST = N - (NS - 1) * RT


@functools.cache
def _build_agg():
    mesh = plsc.VectorSubcoreMesh(core_axis_name="c", subcore_axis_name="s")

    def body(h_hbm, src_hbm, dst_hbm, z_hbm, psum_hbm,
             srci, dsti, rows, acc_sh, gsem, ssem):
        cid = lax.axis_index("c")
        sid = lax.axis_index("s")
        gwid = cid * NS + sid

        @pl.when(sid < NS - 1)
        def _():
            pltpu.sync_copy(z_hbm.at[pl.ds(0, RT)],
                            acc_sh.at[pl.ds(sid * RT, RT)])

        @pl.when(sid == NS - 1)
        def _():
            pltpu.sync_copy(z_hbm, acc_sh.at[pl.ds((NS - 1) * RT, RT_LAST)])

        plsc.subcore_barrier()

        def gather_start(j, r):
            pltpu.async_copy(h_hbm.at[srci.at[j]], rows.at[r], gsem.at[r])

        def gather_wait(j, r):
            pltpu.make_async_copy(h_hbm.at[srci.at[j]], rows.at[r],
                                  gsem.at[r]).wait()

        def scat_start(j, r):
            pltpu.async_copy(rows.at[r], acc_sh.at[dsti.at[j]], ssem.at[r],
                             add=True)

        def scat_wait(j, r):
            pltpu.make_async_copy(rows.at[r], acc_sh.at[dsti.at[j]],
                                  ssem.at[r]).wait()

        @pl.loop(0, NSB)
        def _(s):
            pltpu.sync_copy(src_hbm.at[gwid, s], srci)
            pltpu.sync_copy(dst_hbm.at[gwid, s], dsti)
            for j in range(SB):
                r = j % DEPTH
                if j >= DEPTH:
                    scat_wait(j - DEPTH, r)
                gather_start(j, r)
                if j >= 1:
                    rp = (j - 1) % DEPTH
                    gather_wait(j - 1, rp)
                    scat_start(j - 1, rp)
            rl = (SB - 1) % DEPTH
            gather_wait(SB - 1, rl)
            scat_start(SB - 1, rl)
            for j in range(SB - DEPTH, SB):
                scat_wait(j, j % DEPTH)

        plsc.subcore_barrier()

        @pl.when(sid < NS - 1)
        def _():
            b = sid * RT
            pltpu.sync_copy(acc_sh.at[pl.ds(b, RT)],
                            psum_hbm.at[pl.ds(cid * N + b, RT)])

        @pl.when(sid == NS - 1)
        def _():
            b = (NS - 1) * RT
            pltpu.sync_copy(acc_sh.at[pl.ds(b, RT_LAST)],
                            psum_hbm.at[pl.ds(cid * N + b, RT_LAST)])

    return pl.kernel(
        body,
        out_type=jax.ShapeDtypeStruct((NC * N, D), jnp.float32),
        mesh=mesh,
        scratch_types=[
            pltpu.VMEM((SB, CHUNK), jnp.int32),
            pltpu.VMEM((SB, CHUNK), jnp.int32),
            pltpu.VMEM((DEPTH, CHUNK, D), jnp.float32),
            pltpu.VMEM_SHARED((NPAD, D), jnp.float32),
            pltpu.SemaphoreType.DMA((DEPTH,)),
            pltpu.SemaphoreType.DMA((DEPTH,)),
        ],
    )


@functools.cache
def _build_counts():
    mesh = plsc.VectorSubcoreMesh(core_axis_name="c", subcore_axis_name="s")

    def body(dst_hbm, z_hbm, ones_hbm, pcnt_hbm, dsti, ones_v, cnt_sh, sem):
        cid = lax.axis_index("c")
        sid = lax.axis_index("s")
        gwid = cid * NS + sid

        @pl.when(sid < NS - 1)
        def _():
            pltpu.sync_copy(z_hbm.at[pl.ds(0, RT)],
                            cnt_sh.at[pl.ds(sid * RT, RT)])

        @pl.when(sid == NS - 1)
        def _():
            pltpu.sync_copy(z_hbm, cnt_sh.at[pl.ds((NS - 1) * RT, RT_LAST)])

        pltpu.sync_copy(ones_hbm, ones_v)
        plsc.subcore_barrier()

        @pl.loop(0, NSB)
        def _(s):
            pltpu.sync_copy(dst_hbm.at[gwid, s], dsti)
            for j in range(SB):
                pltpu.async_copy(ones_v, cnt_sh.at[dsti.at[j]], sem, add=True)
            for j in range(SB):
                pltpu.make_async_copy(ones_v, cnt_sh.at[dsti.at[j]],
                                      sem).wait()

        plsc.subcore_barrier()

        @pl.when(sid < NS - 1)
        def _():
            b = sid * RT
            pltpu.sync_copy(cnt_sh.at[pl.ds(b, RT)],
                            pcnt_hbm.at[pl.ds(cid * N + b, RT)])

        @pl.when(sid == NS - 1)
        def _():
            b = (NS - 1) * RT
            pltpu.sync_copy(cnt_sh.at[pl.ds(b, RT_LAST)],
                            pcnt_hbm.at[pl.ds(cid * N + b, RT_LAST)])

    return pl.kernel(
        body,
        out_type=jax.ShapeDtypeStruct((NC * N, D), jnp.float32),
        mesh=mesh,
        scratch_types=[
            pltpu.VMEM((SB, CHUNK), jnp.int32),
            pltpu.VMEM((CHUNK, D), jnp.float32),
            pltpu.VMEM_SHARED((NPAD, D), jnp.float32),
            pltpu.SemaphoreType.DMA,
        ],
    )


BN = 2000


@functools.cache
def _build_linear(relu: bool):
    def body(h_ref, ps_ref, pc_ref, wt_ref, b_ref, o_ref):
        sums = ps_ref[0] + ps_ref[1]
        cnt = pc_ref[0][:, 0:1] + pc_ref[1][:, 0:1]
        row0 = pl.program_id(0) * BN
        rows = jax.lax.broadcasted_iota(jnp.int32, (BN, 1), 0) + row0
        cnt = cnt - jnp.where(rows < PAD, 1.0, 0.0)
        h_n = sums / jnp.maximum(cnt, 1.0)
        acc = jnp.dot(h_ref[...], wt_ref[0:D, :],
                      preferred_element_type=jnp.float32)
        acc += jnp.dot(h_n, wt_ref[D:2 * D, :],
                       preferred_element_type=jnp.float32)
        acc += b_ref[...]
        if relu:
            acc = jnp.maximum(acc, 0.0)
        o_ref[...] = acc

    return pl.pallas_call(
        body,
        grid=(N // BN,),
        in_specs=[
            pl.BlockSpec((BN, D), lambda i: (i, 0)),
            pl.BlockSpec((NC, BN, D), lambda i: (0, i, 0)),
            pl.BlockSpec((NC, BN, D), lambda i: (0, i, 0)),
            pl.BlockSpec((2 * D, D), lambda i: (0, 0)),
            pl.BlockSpec((1, D), lambda i: (0, 0)),
        ],
        out_specs=pl.BlockSpec((BN, D), lambda i: (i, 0)),
        out_shape=jax.ShapeDtypeStruct((N, D), jnp.float32),
    )


def kernel(x, edge_index, W1, b1, W2, b2):
    pad_src = N + (jnp.arange(PAD, dtype=jnp.int32) % (NPAD - N))
    pad_dst = jnp.arange(PAD, dtype=jnp.int32)
    src = jnp.concatenate(
        [edge_index[0].reshape(NW, E // NW),
         pad_src.reshape(NW, PAD // NW)], axis=1
    ).reshape(NW, NSB, SB, CHUNK)
    dst = jnp.concatenate(
        [edge_index[1].reshape(NW, E // NW),
         pad_dst.reshape(NW, PAD // NW)], axis=1
    ).reshape(NW, NSB, SB, CHUNK)
    zeros = jnp.zeros((RT_LAST, D), jnp.float32)
    ones = jnp.ones((CHUNK, D), jnp.float32)
    w1t = W1.T
    w2t = W2.T
    b1r = b1.reshape(1, D)
    b2r = b2.reshape(1, D)

    zrow = jnp.zeros((NPAD - N, D), jnp.float32)
    pc = _build_counts()(dst, zeros, ones).reshape(NC, N, D)
    ps1 = _build_agg()(jnp.concatenate([x, zrow]), src, dst,
                       zeros).reshape(NC, N, D)
    h1 = _build_linear(True)(x, ps1, pc, w1t, b1r)
    ps2 = _build_agg()(jnp.concatenate([h1, zrow]), src, dst,
                       zeros).reshape(NC, N, D)
    return _build_linear(False)(h1, ps2, pc, w2t, b2r)
